# Initial kernel scaffold; baseline (speedup 1.0000x reference)
import functools

import jax
import jax.numpy as jnp
from jax import lax
from jax.experimental import pallas as pl
from jax.experimental.pallas import tpu as pltpu
from jax.experimental.pallas import tpu_sc as plsc

N = 100000
E = 6400000
NUM_GRAPHS = 128
HID = 32
BN_EPS = 1e-5

NC = 2
NS = 16
HALF = N // NC
ROWS_PER_TILE = HALF // NS
TRASH = 1024
ACC_ROWS = HALF + TRASH
EDGES_PER_TILE = E // NS
CHUNK = 1000
NCHUNK = EDGES_PER_TILE // CHUNK


def _sc_aggregate(h):
  D = h.shape[1]
  mesh = plsc.VectorSubcoreMesh(core_axis_name="c", subcore_axis_name="s")

  @functools.partial(
      pl.kernel,
      out_type=jax.ShapeDtypeStruct((N, D), jnp.float32),
      mesh=mesh,
      scratch_types=[
          pltpu.VMEM_SHARED((ACC_ROWS, D), jnp.float32),
          pltpu.VMEM((CHUNK,), jnp.int32),
          pltpu.VMEM((CHUNK,), jnp.int32),
          pltpu.VMEM((CHUNK,), jnp.int32),
          pltpu.VMEM((CHUNK, D), jnp.float32),
          pltpu.SemaphoreType.DMA,
      ],
  )
  def agg_kernel(h_hbm, src_hbm, dst_hbm, out_hbm,
                 acc, srcbuf, dstbuf, locbuf, rows, gsem):
    c = lax.axis_index("c")
    s = lax.axis_index("s")
    node_base = c * HALF + s * ROWS_PER_TILE
    local_base = s * ROWS_PER_TILE
    pltpu.sync_copy(h_hbm.at[pl.ds(node_base, ROWS_PER_TILE)],
                    acc.at[pl.ds(local_base, ROWS_PER_TILE)])
    plsc.subcore_barrier()

    range_base = c * HALF
    lane = lax.broadcasted_iota(jnp.int32, (16,), 0)
    edge_base = s * EDGES_PER_TILE

    def chunk_body(g, _):
      eb = edge_base + g * CHUNK
      pltpu.sync_copy(src_hbm.at[pl.ds(eb, CHUNK)], srcbuf)
      pltpu.sync_copy(dst_hbm.at[pl.ds(eb, CHUNK)], dstbuf)
      gather = pltpu.async_copy(h_hbm.at[srcbuf], rows, gsem)

      def loc_body(i, _):
        v = dstbuf[pl.ds(i * 16, 16)]
        rel = v - range_base
        ok = (rel >= 0) & (rel < HALF)
        tr = HALF + ((lane + i * 16) & (TRASH - 1))
        locbuf[pl.ds(i * 16, 16)] = jnp.where(ok, rel, tr)
        return 0

      lax.fori_loop(0, CHUNK // 16, loc_body, 0)
      gather.wait()
      pltpu.sync_copy(rows, acc.at[locbuf], add=True)
      return 0

    lax.fori_loop(0, NCHUNK, chunk_body, 0)
    plsc.subcore_barrier()
    pltpu.sync_copy(acc.at[pl.ds(local_base, ROWS_PER_TILE)],
                    out_hbm.at[pl.ds(node_base, ROWS_PER_TILE)])

  return agg_kernel


MLP_BLK = 5000
MLP_NBLK = N // MLP_BLK


def _mlp_body(relu_out, z_ref, w1_ref, b1_ref, g_ref, bta_ref, w2_ref, b2_ref,
              out_ref, stats):
  p = pl.program_id(0)
  b = pl.program_id(1)

  @pl.when((p == 0) & (b == 0))
  def _():
    stats[...] = jnp.zeros_like(stats)

  z = z_ref[...]

  @pl.when(p == 0)
  def _():
    stats[0, :] += jnp.sum(z, axis=0)

  @pl.when(p > 0)
  def _():
    hid = jnp.dot(z, w1_ref[...], preferred_element_type=jnp.float32)
    mu = (jnp.dot(stats[0:1, :], w1_ref[...],
                  preferred_element_type=jnp.float32) / N) + b1_ref[...]
    dev = hid + b1_ref[...] - mu

    @pl.when(p == 1)
    def _():
      stats[1, :] += jnp.sum(dev * dev, axis=0)

    @pl.when(p == 2)
    def _():
      rstd = lax.rsqrt(stats[1:2, :] / N + BN_EPS)
      hnorm = jax.nn.relu(dev * rstd * g_ref[...] + bta_ref[...])
      o = jnp.dot(hnorm, w2_ref[...],
                  preferred_element_type=jnp.float32) + b2_ref[...]
      if relu_out:
        o = jax.nn.relu(o)
      out_ref[...] = o


def _tc_mlp(z, w1, b1, g, bta, w2, b2, relu_out):
  row = lambda v: v.reshape(1, -1)
  spec_full = lambda shp: pl.BlockSpec(shp, lambda p, b: (0, 0))
  return pl.pallas_call(
      functools.partial(_mlp_body, relu_out),
      grid=(3, MLP_NBLK),
      in_specs=[
          pl.BlockSpec((MLP_BLK, HID), lambda p, b: (b, 0)),
          spec_full((HID, HID)),
          spec_full((1, HID)),
          spec_full((1, HID)),
          spec_full((1, HID)),
          spec_full((HID, HID)),
          spec_full((1, HID)),
      ],
      out_specs=pl.BlockSpec((MLP_BLK, HID), lambda p, b: (b, 0)),
      out_shape=jax.ShapeDtypeStruct((N, HID), jnp.float32),
      scratch_shapes=[pltpu.VMEM((8, HID), jnp.float32)],
  )(z, w1, row(b1), row(g), row(bta), w2, row(b2))


def _pool_body(h_ref, batch_ref, fw1_ref, fb1_ref, fw2_ref, fb2_ref,
               out_ref, gacc):
  b = pl.program_id(0)

  @pl.when(b == 0)
  def _():
    gacc[...] = jnp.zeros_like(gacc)

  ids = batch_ref[...]
  onehot = (lax.broadcasted_iota(jnp.int32, (NUM_GRAPHS, MLP_BLK), 0)
            == ids[None, :]).astype(jnp.float32)
  gacc[...] += jnp.dot(onehot, h_ref[...], preferred_element_type=jnp.float32)

  @pl.when(b == MLP_NBLK - 1)
  def _():
    t = jax.nn.relu(jnp.dot(gacc[...], fw1_ref[...],
                            preferred_element_type=jnp.float32) + fb1_ref[...])
    out_ref[...] = jnp.dot(t, fw2_ref[...],
                           preferred_element_type=jnp.float32) + fb2_ref[...]


def _tc_pool_head(h, batch, fw1, fb1, fw2, fb2):
  fw2p = jnp.zeros((HID, 8), jnp.float32).at[:, 0].set(fw2[:, 0])
  fb2p = jnp.zeros((1, 8), jnp.float32).at[0, 0].set(fb2[0])
  out = pl.pallas_call(
      _pool_body,
      grid=(MLP_NBLK,),
      in_specs=[
          pl.BlockSpec((MLP_BLK, HID), lambda b: (b, 0)),
          pl.BlockSpec((MLP_BLK,), lambda b: (b,)),
          pl.BlockSpec((HID, HID), lambda b: (0, 0)),
          pl.BlockSpec((1, HID), lambda b: (0, 0)),
          pl.BlockSpec((HID, 8), lambda b: (0, 0)),
          pl.BlockSpec((1, 8), lambda b: (0, 0)),
      ],
      out_specs=pl.BlockSpec((NUM_GRAPHS, 8), lambda b: (0, 0)),
      out_shape=jax.ShapeDtypeStruct((NUM_GRAPHS, 8), jnp.float32),
      scratch_shapes=[pltpu.VMEM((NUM_GRAPHS, HID), jnp.float32)],
  )(h, batch, fw1, fb1.reshape(1, -1), fw2p, fb2p)
  return out[:, 0]


def kernel(x, edge_index, batch, params):
  src = edge_index[0]
  dst = edge_index[1]
  h = jnp.pad(x, ((0, 0), (0, HID - x.shape[1])))
  layers = params["gin"]
  num_layers = len(layers)
  for i, (w1, b1, g, bta, w2, b2) in enumerate(layers):
    w1p = w1 if w1.shape[0] == HID else jnp.pad(
        w1, ((0, HID - w1.shape[0]), (0, 0)))
    z = _sc_aggregate(h)(h, src, dst)
    h = _tc_mlp(z, w1p, b1, g, bta, w2, b2, relu_out=(i < num_layers - 1))
  fw1, fb1 = params["fc1"]
  fw2, fb2 = params["fc2"]
  return _tc_pool_head(h, batch, fw1, fb1, fw2, fb2)


# speedup vs baseline: 15.8819x; 15.8819x over previous
import functools

import jax
import jax.numpy as jnp
from jax import lax
from jax.experimental import pallas as pl
from jax.experimental.pallas import tpu as pltpu
from jax.experimental.pallas import tpu_sc as plsc

N = 100000
E = 6400000
NUM_GRAPHS = 128
HID = 32
BN_EPS = 1e-5

NC = 2
NS = 16
HALF = N // NC
ROWS_PER_TILE = HALF // NS
TRASH = 512
ACC_ROWS = HALF + TRASH
EDGES_PER_TILE = E // NS
SUBW = 128
SUBN = 5
CHUNK = SUBN * SUBW
NCHUNK = EDGES_PER_TILE // CHUNK
ROWS128_PER_TILE = EDGES_PER_TILE // SUBW


def _sc_aggregate(h):
  D = h.shape[1]
  mesh = plsc.VectorSubcoreMesh(core_axis_name="c", subcore_axis_name="s",
                                num_cores=NC, num_subcores=NS)

  @functools.partial(
      pl.kernel,
      out_type=jax.ShapeDtypeStruct((N, D), jnp.float32),
      mesh=mesh,
      compiler_params=pltpu.CompilerParams(use_tc_tiling_on_sc=False),
      scratch_types=[
          pltpu.VMEM_SHARED((ACC_ROWS, D), jnp.float32),
          pltpu.VMEM((SUBN, SUBW), jnp.int32),
          pltpu.VMEM((SUBN, SUBW), jnp.int32),
          pltpu.VMEM((SUBN, SUBW), jnp.int32),
          pltpu.VMEM((CHUNK, D), jnp.float32),
          pltpu.SemaphoreType.DMA,
          pltpu.SemaphoreType.DMA,
      ],
  )
  def agg_kernel(h_hbm, src_hbm, dst_hbm, out_hbm,
                 acc, srcbuf, dstbuf, locbuf, rows, gsem, ssem):
    c = lax.axis_index("c")
    s = lax.axis_index("s")
    node_base = c * HALF + s * ROWS_PER_TILE
    local_base = s * ROWS_PER_TILE
    pltpu.sync_copy(h_hbm.at[pl.ds(node_base, ROWS_PER_TILE)],
                    acc.at[pl.ds(local_base, ROWS_PER_TILE)])
    plsc.subcore_barrier()

    range_base = c * HALF
    lane = lax.broadcasted_iota(jnp.int32, (16,), 0)
    row_base = s * ROWS128_PER_TILE

    def chunk_body(g, _):
      rb = row_base + g * SUBN
      pltpu.sync_copy(src_hbm.at[pl.ds(rb, SUBN)], srcbuf)
      pltpu.sync_copy(dst_hbm.at[pl.ds(rb, SUBN)], dstbuf)
      for j in range(SUBN):
        pltpu.async_copy(h_hbm.at[srcbuf.at[j]],
                         rows.at[pl.ds(j * SUBW, SUBW)], gsem)

      def loc_body(i, _):
        j = i // (SUBW // 16)
        k = lax.rem(i, SUBW // 16)
        v = dstbuf[j, pl.ds(k * 16, 16)]
        rel = v - range_base
        ok = (rel >= 0) & (rel < HALF)
        tr = HALF + ((lane + i * 16) & (TRASH - 1))
        locbuf[j, pl.ds(k * 16, 16)] = jnp.where(ok, rel, tr)
        return 0

      lax.fori_loop(0, CHUNK // 16, loc_body, 0)
      for j in range(SUBN):
        pltpu.make_async_copy(h_hbm.at[srcbuf.at[j]],
                              rows.at[pl.ds(j * SUBW, SUBW)], gsem).wait()
      for j in range(SUBN):
        pltpu.async_copy(rows.at[pl.ds(j * SUBW, SUBW)],
                         acc.at[locbuf.at[j]], ssem, add=True)
      for j in range(SUBN):
        pltpu.make_async_copy(rows.at[pl.ds(j * SUBW, SUBW)],
                              acc.at[locbuf.at[j]], ssem).wait()
      return 0

    lax.fori_loop(0, NCHUNK, chunk_body, 0)
    plsc.subcore_barrier()
    pltpu.sync_copy(acc.at[pl.ds(local_base, ROWS_PER_TILE)],
                    out_hbm.at[pl.ds(node_base, ROWS_PER_TILE)])

  return agg_kernel


MLP_BLK = 5000
MLP_NBLK = N // MLP_BLK


def _mlp_body(relu_out, z_ref, w1_ref, b1_ref, g_ref, bta_ref, w2_ref, b2_ref,
              out_ref, stats):
  p = pl.program_id(0)
  b = pl.program_id(1)

  @pl.when((p == 0) & (b == 0))
  def _():
    stats[...] = jnp.zeros_like(stats)

  hid = jnp.dot(z_ref[...], w1_ref[...],
                preferred_element_type=jnp.float32) + b1_ref[...]

  @pl.when(p == 0)
  def _():
    stats[0, :] += jnp.sum(hid, axis=0)

  @pl.when(p > 0)
  def _():
    dev = hid - stats[0:1, :] / N

    @pl.when(p == 1)
    def _():
      stats[1, :] += jnp.sum(dev * dev, axis=0)

    @pl.when(p == 2)
    def _():
      sd = jnp.sqrt(stats[1:2, :] / N + BN_EPS)
      hnorm = jax.nn.relu(dev / sd * g_ref[...] + bta_ref[...])
      o = jnp.dot(hnorm, w2_ref[...],
                  preferred_element_type=jnp.float32) + b2_ref[...]
      if relu_out:
        o = jax.nn.relu(o)
      out_ref[...] = o


def _tc_mlp(z, w1, b1, g, bta, w2, b2, relu_out):
  row = lambda v: v.reshape(1, -1)
  spec_full = lambda shp: pl.BlockSpec(shp, lambda p, b: (0, 0))
  return pl.pallas_call(
      functools.partial(_mlp_body, relu_out),
      grid=(3, MLP_NBLK),
      in_specs=[
          pl.BlockSpec((MLP_BLK, HID), lambda p, b: (b, 0)),
          spec_full((HID, HID)),
          spec_full((1, HID)),
          spec_full((1, HID)),
          spec_full((1, HID)),
          spec_full((HID, HID)),
          spec_full((1, HID)),
      ],
      out_specs=pl.BlockSpec((MLP_BLK, HID), lambda p, b: (b, 0)),
      out_shape=jax.ShapeDtypeStruct((N, HID), jnp.float32),
      scratch_shapes=[pltpu.VMEM((8, HID), jnp.float32)],
  )(z, w1, row(b1), row(g), row(bta), w2, row(b2))


def _pool_body(h_ref, batch_ref, fw1_ref, fb1_ref, fw2_ref, fb2_ref,
               out_ref, gacc):
  b = pl.program_id(0)

  @pl.when(b == 0)
  def _():
    gacc[...] = jnp.zeros_like(gacc)

  ids = batch_ref[0, 0, :]
  onehot = (lax.broadcasted_iota(jnp.int32, (NUM_GRAPHS, MLP_BLK), 0)
            == ids[None, :]).astype(jnp.float32)
  gacc[...] += jnp.dot(onehot, h_ref[...], preferred_element_type=jnp.float32,
                 precision=lax.Precision.HIGHEST)

  @pl.when(b == MLP_NBLK - 1)
  def _():
    t = jax.nn.relu(jnp.dot(gacc[...], fw1_ref[...],
                            preferred_element_type=jnp.float32)
                    + fb1_ref[...])
    out_ref[...] = jnp.dot(t, fw2_ref[...],
                           preferred_element_type=jnp.float32) + fb2_ref[...]


def _tc_pool_head(h, batch, fw1, fb1, fw2, fb2):
  fw2p = jnp.zeros((HID, 8), jnp.float32).at[:, 0].set(fw2[:, 0])
  fb2p = jnp.zeros((1, 8), jnp.float32).at[0, 0].set(fb2[0])
  out = pl.pallas_call(
      _pool_body,
      grid=(MLP_NBLK,),
      in_specs=[
          pl.BlockSpec((MLP_BLK, HID), lambda b: (b, 0)),
          pl.BlockSpec((1, 1, MLP_BLK), lambda b: (b, 0, 0)),
          pl.BlockSpec((HID, HID), lambda b: (0, 0)),
          pl.BlockSpec((1, HID), lambda b: (0, 0)),
          pl.BlockSpec((HID, 8), lambda b: (0, 0)),
          pl.BlockSpec((1, 8), lambda b: (0, 0)),
      ],
      out_specs=pl.BlockSpec((NUM_GRAPHS, 8), lambda b: (0, 0)),
      out_shape=jax.ShapeDtypeStruct((NUM_GRAPHS, 8), jnp.float32),
      scratch_shapes=[pltpu.VMEM((NUM_GRAPHS, HID), jnp.float32)],
  )(h, batch.reshape(MLP_NBLK, 1, MLP_BLK), fw1, fb1.reshape(1, -1),
    fw2p, fb2p)
  return out[:, 0]


def kernel(x, edge_index, batch, params):
  src = edge_index[0]
  dst = edge_index[1]
  h = jnp.pad(x, ((0, 0), (0, HID - x.shape[1])))
  layers = params["gin"]
  num_layers = len(layers)
  for i, (w1, b1, g, bta, w2, b2) in enumerate(layers):
    w1p = w1 if w1.shape[0] == HID else jnp.pad(
        w1, ((0, HID - w1.shape[0]), (0, 0)))
    z = _sc_aggregate(h)(h, src.reshape(E // SUBW, SUBW),
                         dst.reshape(E // SUBW, SUBW))
    h = _tc_mlp(z, w1p, b1, g, bta, w2, b2, relu_out=(i < num_layers - 1))
  fw1, fb1 = params["fc1"]
  fw2, fb2 = params["fc2"]
  return _tc_pool_head(h, batch, fw1, fb1, fw2, fb2)
